# baseline (speedup 1.0000x reference)
import jax
import jax.numpy as jnp
from jax import lax
from jax.experimental import pallas as pl
from jax.experimental.pallas import tpu as pltpu
from jax.experimental.pallas import tpu_sc as plsc

NUM_CARDS = 100000
CARD_ID_DIM = 64
ROW_I32 = CARD_ID_DIM // 2
HIDDEN_DIM = 128
BATCH = 4096
SEQ_LEN = 200
FEAT_EXTRA = 11
TOTAL = BATCH * SEQ_LEN

NC = 2
NS = 16
NW = NC * NS
CHUNK = 128

K = 4
NB = BATCH // 512
NB_K = NB // K
BPK = BATCH // K
ROWS_K = TOTAL // K
PER_W = ROWS_K // NW
NCHUNK = PER_W // CHUNK
SROWS = 8

BB = 512
SB = 8
QUAR = BB // 4


def _make_gather_body(kofs):
    def _gather_body(ids_hbm, table_hbm, out_hbm, sids, idx_v, rows_v, sem_g):
        _gather_impl(kofs, ids_hbm, table_hbm, out_hbm, sids, idx_v, rows_v, sem_g)
    return _gather_body


def _gather_impl(kofs, ids_hbm, table_hbm, out_hbm, sids, idx_v, rows_v, sem_g):
    wid = lax.axis_index("s") * NC + lax.axis_index("c")
    base = wid * PER_W
    s_lo = jnp.minimum(base >> 10, SEQ_LEN - SROWS)

    pltpu.sync_copy(ids_hbm.at[pl.ds(s_lo, SROWS), pl.ds(kofs, BPK)], sids)

    lanes = lax.iota(jnp.int32, 16)
    phase = lanes & 3
    idx_base = lanes >> 2
    dn = lax.GatherDimensionNumbers(
        offset_dims=(), collapsed_slice_dims=(0,), start_index_map=(0,)
    )

    def rgather(vec, idx):
        return lax.gather(
            vec, idx[:, None], dn, (1,),
            mode=lax.GatherScatterMode.PROMISE_IN_BOUNDS,
        )

    def build(i, _):
        flat = base + i * CHUNK
        srel = (flat >> 10) - s_lo
        rem_base = flat & 1023
        ibl = rem_base >> 9
        j0 = (rem_base >> 2) & 127
        for grp in range(2):
            srcs = [
                sids[srel, pl.ds(ibl * 512 + h * 128 + j0 + grp * 16, 16)]
                for h in range(4)
            ]
            for q in range(4):
                idxq = idx_base + q * 4
                g0 = rgather(srcs[0], idxq)
                g1 = rgather(srcs[1], idxq)
                g2 = rgather(srcs[2], idxq)
                g3 = rgather(srcs[3], idxq)
                vals = jnp.where(
                    phase == 0, g0,
                    jnp.where(phase == 1, g1, jnp.where(phase == 2, g2, g3)),
                )
                idx_v[i, pl.ds(grp * 64 + q * 16, 16)] = vals
        return ()

    lax.fori_loop(0, NCHUNK, build, (), unroll=False)

    def body(i, _):
        pltpu.async_copy(table_hbm.at[idx_v.at[i]], rows_v, sem_g).wait()
        pltpu.sync_copy(rows_v, out_hbm.at[pl.ds(base + i * CHUNK, CHUNK)])
        return ()

    lax.fori_loop(0, NCHUNK, body, (), unroll=False)


def _sc_gather(k, ids2, table_p):
    mesh = plsc.VectorSubcoreMesh(
        core_axis_name="c", subcore_axis_name="s", num_cores=NC, num_subcores=NS
    )
    return pl.kernel(
        _make_gather_body(k * BPK),
        out_type=jax.ShapeDtypeStruct((ROWS_K, ROW_I32), jnp.int32),
        mesh=mesh,
        compiler_params=pltpu.CompilerParams(use_tc_tiling_on_sc=False),
        scratch_types=[
            pltpu.VMEM((SROWS, BPK), jnp.int32),
            pltpu.VMEM((NCHUNK, CHUNK), jnp.int32),
            pltpu.VMEM((CHUNK, ROW_I32), jnp.int32),
            pltpu.SemaphoreType.DMA,
        ],
    )(ids2, table_p)


def _mm_body(g_ref, f_ref, we_ref, wo_ref, wf_ref, b_ref, o_ref):
    bvec = b_ref[...]
    we = we_ref[...]
    wo = wo_ref[...]
    wf = wf_ref[...]
    for s in range(SB):
        gi = g_ref[s]
        lo = lax.bitcast_convert_type(gi << 16, jnp.float32)
        hi = lax.bitcast_convert_type(gi & (-65536), jnp.float32)
        fts = f_ref[:, s, :]
        of = lax.dot_general(
            fts, wf,
            dimension_numbers=(((0,), (0,)), ((), ())),
            preferred_element_type=jnp.float32,
        )
        for h in range(4):
            sl = slice(h * ROW_I32, (h + 1) * ROW_I32)
            oh = jnp.dot(lo[:, sl], we, preferred_element_type=jnp.float32)
            oh = oh + jnp.dot(hi[:, sl], wo, preferred_element_type=jnp.float32)
            o_ref[h * QUAR:(h + 1) * QUAR, s, :] = (
                oh + of[h * QUAR:(h + 1) * QUAR] + bvec
            )


def _mm_body_acc(g_ref, f_ref, we_ref, wo_ref, wf_ref, b_ref, _prev_ref, o_ref):
    _mm_body(g_ref, f_ref, we_ref, wo_ref, wf_ref, b_ref, o_ref)


def _tc_project_chunk(k, g3, ft, we, wo, wf_t, b2, prev):
    grid = (NB_K, SEQ_LEN // SB)
    in_specs = [
        pl.BlockSpec((SB, QUAR, HIDDEN_DIM), lambda ib, isq: (isq, ib, 0)),
        pl.BlockSpec(
            (FEAT_EXTRA, SB, BB), lambda ib, isq, k=k: (0, isq, k * NB_K + ib)
        ),
        pl.BlockSpec((ROW_I32, HIDDEN_DIM), lambda ib, isq: (0, 0)),
        pl.BlockSpec((ROW_I32, HIDDEN_DIM), lambda ib, isq: (0, 0)),
        pl.BlockSpec((FEAT_EXTRA, HIDDEN_DIM), lambda ib, isq: (0, 0)),
        pl.BlockSpec((1, HIDDEN_DIM), lambda ib, isq: (0, 0)),
    ]
    out_spec = pl.BlockSpec(
        (BB, SB, HIDDEN_DIM), lambda ib, isq, k=k: (k * NB_K + ib, isq, 0)
    )
    out_shape = jax.ShapeDtypeStruct((BATCH, SEQ_LEN, HIDDEN_DIM), jnp.float32)
    if prev is None:
        return pl.pallas_call(
            _mm_body,
            grid=grid,
            in_specs=in_specs,
            out_specs=out_spec,
            out_shape=out_shape,
        )(g3, ft, we, wo, wf_t, b2)
    return pl.pallas_call(
        _mm_body_acc,
        grid=grid,
        in_specs=in_specs + [pl.BlockSpec(memory_space=pl.ANY)],
        out_specs=out_spec,
        out_shape=out_shape,
        input_output_aliases={6: 0},
    )(g3, ft, we, wo, wf_t, b2, prev)


@jax.jit
def _run(ids2, ft, table_p, we, wo, wf_t, b2):
    gs = [_sc_gather(k, ids2, table_p) for k in range(K)]
    out = None
    for k in range(K):
        g3 = gs[k].reshape(SEQ_LEN, BPK // 4, HIDDEN_DIM)
        out = _tc_project_chunk(k, g3, ft, we, wo, wf_t, b2, out)
    return out


def kernel(card_ids, card_features, table, W, b):
    ids2 = card_ids.T.astype(jnp.int32)
    table_p = lax.bitcast_convert_type(
        table.astype(jnp.bfloat16).reshape(NUM_CARDS, ROW_I32, 2), jnp.int32
    )
    ft = jnp.transpose(card_features, (2, 1, 0))
    wid_t = W[:, :CARD_ID_DIM].T
    we = wid_t[0::2]
    wo = wid_t[1::2]
    wf_t = W[:, CARD_ID_DIM:].T
    b2 = b.reshape(1, HIDDEN_DIM)
    return _run(ids2, ft, table_p, we, wo, wf_t, b2)


# speedup vs baseline: 1.2387x; 1.2387x over previous
import jax
import jax.numpy as jnp
from jax import lax
from jax.experimental import pallas as pl
from jax.experimental.pallas import tpu as pltpu
from jax.experimental.pallas import tpu_sc as plsc

NUM_CARDS = 100000
CARD_ID_DIM = 64
ROW_I32 = CARD_ID_DIM // 2
HIDDEN_DIM = 128
BATCH = 4096
SEQ_LEN = 200
FEAT_EXTRA = 11
TOTAL = BATCH * SEQ_LEN

NC = 2
NS = 16
NW = NC * NS
CHUNK = 128

K = 4
NB = BATCH // 512
NB_K = NB // K
BPK = BATCH // K
ROWS_K = TOTAL // K
PER_W = ROWS_K // NW
NCHUNK = PER_W // CHUNK
SROWS = 8

BB = 512
SB = 8
QUAR = BB // 4


def _make_gather_body(kofs):
    def _gather_body(ids_hbm, table_hbm, out_hbm, sids, idx_v, rows_v, sem_g):
        _gather_impl(kofs, ids_hbm, table_hbm, out_hbm, sids, idx_v, rows_v, sem_g)
    return _gather_body


def _gather_impl(kofs, ids_hbm, table_hbm, out_hbm, sids, idx_v, rows_v, sem_g):
    wid = lax.axis_index("s") * NC + lax.axis_index("c")
    base = wid * PER_W
    s_lo = jnp.minimum(base >> 10, SEQ_LEN - SROWS)

    pltpu.sync_copy(ids_hbm.at[pl.ds(s_lo, SROWS), pl.ds(kofs, BPK)], sids)

    lanes = lax.iota(jnp.int32, 16)
    phase = lanes & 3
    idx_base = lanes >> 2
    dn = lax.GatherDimensionNumbers(
        offset_dims=(), collapsed_slice_dims=(0,), start_index_map=(0,)
    )

    def rgather(vec, idx):
        return lax.gather(
            vec, idx[:, None], dn, (1,),
            mode=lax.GatherScatterMode.PROMISE_IN_BOUNDS,
        )

    def build(i, _):
        flat = base + i * CHUNK
        srel = (flat >> 10) - s_lo
        rem_base = flat & 1023
        ibl = rem_base >> 9
        j0 = (rem_base >> 2) & 127
        for grp in range(2):
            srcs = [
                sids[srel, pl.ds(ibl * 512 + h * 128 + j0 + grp * 16, 16)]
                for h in range(4)
            ]
            for q in range(4):
                idxq = idx_base + q * 4
                g0 = rgather(srcs[0], idxq)
                g1 = rgather(srcs[1], idxq)
                g2 = rgather(srcs[2], idxq)
                g3 = rgather(srcs[3], idxq)
                vals = jnp.where(
                    phase == 0, g0,
                    jnp.where(phase == 1, g1, jnp.where(phase == 2, g2, g3)),
                )
                idx_v[i, pl.ds(grp * 64 + q * 16, 16)] = vals
        return ()

    lax.fori_loop(0, NCHUNK, build, (), unroll=False)

    def body(i, _):
        pltpu.async_copy(table_hbm.at[idx_v.at[i]], rows_v, sem_g).wait()
        pltpu.sync_copy(rows_v, out_hbm.at[pl.ds(base + i * CHUNK, CHUNK)])
        return ()

    lax.fori_loop(0, NCHUNK, body, (), unroll=False)


def _sc_gather(k, ids2, table_p):
    mesh = plsc.VectorSubcoreMesh(
        core_axis_name="c", subcore_axis_name="s", num_cores=NC, num_subcores=NS
    )
    return pl.kernel(
        _make_gather_body(k * BPK),
        out_type=jax.ShapeDtypeStruct((ROWS_K, ROW_I32), jnp.int32),
        mesh=mesh,
        compiler_params=pltpu.CompilerParams(use_tc_tiling_on_sc=False),
        scratch_types=[
            pltpu.VMEM((SROWS, BPK), jnp.int32),
            pltpu.VMEM((NCHUNK, CHUNK), jnp.int32),
            pltpu.VMEM((CHUNK, ROW_I32), jnp.int32),
            pltpu.SemaphoreType.DMA,
        ],
    )(ids2, table_p)


def _mm_body(g_ref, f_ref, we_ref, wo_ref, wf_ref, b_ref, o_ref):
    bvec = b_ref[...]
    we = we_ref[...]
    wo = wo_ref[...]
    wf = wf_ref[...]
    for s in range(SB):
        gi = g_ref[s]
        lo = lax.bitcast_convert_type(gi << 16, jnp.float32)
        hi = lax.bitcast_convert_type(gi & (-65536), jnp.float32)
        fts = f_ref[:, s, :]
        of = lax.dot_general(
            fts, wf,
            dimension_numbers=(((0,), (0,)), ((), ())),
            preferred_element_type=jnp.float32,
        )
        for h in range(4):
            sl = slice(h * ROW_I32, (h + 1) * ROW_I32)
            oh = jnp.dot(lo[:, sl], we, preferred_element_type=jnp.float32)
            oh = oh + jnp.dot(hi[:, sl], wo, preferred_element_type=jnp.float32)
            o_ref[h * QUAR:(h + 1) * QUAR, s, :] = (
                oh + of[h * QUAR:(h + 1) * QUAR] + bvec
            )


def _mm_body_acc(g_ref, f_ref, we_ref, wo_ref, wf_ref, b_ref, _prev_ref, o_ref):
    _mm_body(g_ref, f_ref, we_ref, wo_ref, wf_ref, b_ref, o_ref)


def _tc_project_chunk(k, g3, ft, we, wo, wf_t, b2, prev):
    grid = (NB_K, SEQ_LEN // SB)
    in_specs = [
        pl.BlockSpec((SB, QUAR, HIDDEN_DIM), lambda ib, isq: (isq, ib, 0)),
        pl.BlockSpec(
            (FEAT_EXTRA, SB, BB), lambda ib, isq, k=k: (0, isq, k * NB_K + ib)
        ),
        pl.BlockSpec((ROW_I32, HIDDEN_DIM), lambda ib, isq: (0, 0)),
        pl.BlockSpec((ROW_I32, HIDDEN_DIM), lambda ib, isq: (0, 0)),
        pl.BlockSpec((FEAT_EXTRA, HIDDEN_DIM), lambda ib, isq: (0, 0)),
        pl.BlockSpec((1, HIDDEN_DIM), lambda ib, isq: (0, 0)),
    ]
    out_spec = pl.BlockSpec(
        (BB, SB, HIDDEN_DIM), lambda ib, isq, k=k: (k * NB_K + ib, isq, 0)
    )
    out_shape = jax.ShapeDtypeStruct((BATCH, SEQ_LEN, HIDDEN_DIM), jnp.float32)
    if prev is None:
        return pl.pallas_call(
            _mm_body,
            grid=grid,
            in_specs=in_specs,
            out_specs=out_spec,
            out_shape=out_shape,
        )(g3, ft, we, wo, wf_t, b2)
    return pl.pallas_call(
        _mm_body_acc,
        grid=grid,
        in_specs=in_specs + [pl.BlockSpec(memory_space=pl.ANY)],
        out_specs=out_spec,
        out_shape=out_shape,
        input_output_aliases={6: 0},
    )(g3, ft, we, wo, wf_t, b2, prev)


@jax.jit
def _run(ids2, ft, table_p, we, wo, wf_t, b2):
    gs = [_sc_gather(k, ids2, table_p) for k in range(K)]
    out = None
    for k in range(K):
        g3 = gs[k].reshape(SEQ_LEN, BPK // 4, HIDDEN_DIM)
        out = _tc_project_chunk(k, g3, ft, we, wo, wf_t, b2, out)
    return out


def kernel(card_ids, card_features, table, W, b):
    ids2 = card_ids.T.astype(jnp.int32)
    tt = table.T
    tbits = lax.bitcast_convert_type(
        tt.astype(jnp.bfloat16).astype(jnp.float32), jnp.int32
    )
    words = ((tbits[:ROW_I32] >> 16) & 0xFFFF) | (tbits[ROW_I32:] & (-65536))
    table_p = words.T
    ft = jnp.transpose(card_features, (2, 1, 0))
    wid_t = W[:, :CARD_ID_DIM].T
    we = wid_t[:ROW_I32]
    wo = wid_t[ROW_I32:]
    wf_t = W[:, CARD_ID_DIM:].T
    b2 = b.reshape(1, HIDDEN_DIM)
    return _run(ids2, ft, table_p, we, wo, wf_t, b2)


# speedup vs baseline: 1.3211x; 1.0665x over previous
import jax
import jax.numpy as jnp
from jax import lax
from jax.experimental import pallas as pl
from jax.experimental.pallas import tpu as pltpu
from jax.experimental.pallas import tpu_sc as plsc

NUM_CARDS = 100000
CARD_ID_DIM = 64
ROW_I32 = CARD_ID_DIM // 2
HIDDEN_DIM = 128
BATCH = 4096
SEQ_LEN = 200
FEAT_EXTRA = 11
TOTAL = BATCH * SEQ_LEN

NC = 2
NS = 16
NW = NC * NS
CHUNK = 128

K = 4
NB = BATCH // 512
NB_K = NB // K
BPK = BATCH // K
ROWS_K = TOTAL // K
PER_W = ROWS_K // NW
NCHUNK = PER_W // CHUNK
SROWS = 8

BB = 512
SB = 8
QUAR = BB // 4


def _make_gather_body(kofs):
    def _gather_body(ids_hbm, table_hbm, out_hbm, sids, idx_v, rows_v, sem_g):
        _gather_impl(kofs, ids_hbm, table_hbm, out_hbm, sids, idx_v, rows_v, sem_g)
    return _gather_body


def _gather_impl(kofs, ids_hbm, table_hbm, out_hbm, sids, idx_v, rows_v, sem_g):
    wid = lax.axis_index("s") * NC + lax.axis_index("c")
    base = wid * PER_W
    s_lo = jnp.minimum(base >> 10, SEQ_LEN - SROWS)

    pltpu.sync_copy(ids_hbm.at[pl.ds(s_lo, SROWS), pl.ds(kofs, BPK)], sids)

    lanes = lax.iota(jnp.int32, 16)
    phase = lanes & 3
    idx_base = lanes >> 2
    dn = lax.GatherDimensionNumbers(
        offset_dims=(), collapsed_slice_dims=(0,), start_index_map=(0,)
    )

    def rgather(vec, idx):
        return lax.gather(
            vec, idx[:, None], dn, (1,),
            mode=lax.GatherScatterMode.PROMISE_IN_BOUNDS,
        )

    def build(i, _):
        flat = base + i * CHUNK
        srel = (flat >> 10) - s_lo
        rem_base = flat & 1023
        ibl = rem_base >> 9
        j0 = (rem_base >> 2) & 127
        for grp in range(2):
            srcs = [
                sids[srel, pl.ds(ibl * 512 + h * 128 + j0 + grp * 16, 16)]
                for h in range(4)
            ]
            for q in range(4):
                idxq = idx_base + q * 4
                g0 = rgather(srcs[0], idxq)
                g1 = rgather(srcs[1], idxq)
                g2 = rgather(srcs[2], idxq)
                g3 = rgather(srcs[3], idxq)
                vals = jnp.where(
                    phase == 0, g0,
                    jnp.where(phase == 1, g1, jnp.where(phase == 2, g2, g3)),
                )
                idx_v[i, pl.ds(grp * 64 + q * 16, 16)] = vals
        return ()

    lax.fori_loop(0, NCHUNK, build, (), unroll=False)

    def body(i, _):
        pltpu.async_copy(table_hbm.at[idx_v.at[i]], rows_v, sem_g).wait()
        pltpu.sync_copy(rows_v, out_hbm.at[pl.ds(base + i * CHUNK, CHUNK)])
        return ()

    lax.fori_loop(0, NCHUNK, body, (), unroll=False)


def _sc_gather(k, ids2, table_p):
    mesh = plsc.VectorSubcoreMesh(
        core_axis_name="c", subcore_axis_name="s", num_cores=NC, num_subcores=NS
    )
    return pl.kernel(
        _make_gather_body(k * BPK),
        out_type=jax.ShapeDtypeStruct((ROWS_K, ROW_I32), jnp.int32),
        mesh=mesh,
        compiler_params=pltpu.CompilerParams(use_tc_tiling_on_sc=False),
        scratch_types=[
            pltpu.VMEM((SROWS, BPK), jnp.int32),
            pltpu.VMEM((NCHUNK, CHUNK), jnp.int32),
            pltpu.VMEM((CHUNK, ROW_I32), jnp.int32),
            pltpu.SemaphoreType.DMA,
        ],
    )(ids2, table_p)


def _mm_body(g_ref, f_ref, we_ref, wo_ref, wf_ref, b_ref, o_ref):
    bvec = b_ref[...]
    we = we_ref[...]
    wo = wo_ref[...]
    wf = wf_ref[...]
    for s in range(SB):
        gi = g_ref[s]
        lo = lax.bitcast_convert_type(gi << 16, jnp.float32)
        hi = lax.bitcast_convert_type(gi & (-65536), jnp.float32)
        fts = f_ref[:, s, :]
        of = lax.dot_general(
            fts, wf,
            dimension_numbers=(((0,), (0,)), ((), ())),
            preferred_element_type=jnp.float32,
        )
        for h in range(4):
            sl = slice(h * ROW_I32, (h + 1) * ROW_I32)
            oh = jnp.dot(lo[:, sl], we, preferred_element_type=jnp.float32)
            oh = oh + jnp.dot(hi[:, sl], wo, preferred_element_type=jnp.float32)
            o_ref[h * QUAR:(h + 1) * QUAR, s, :] = (
                oh + of[h * QUAR:(h + 1) * QUAR] + bvec
            )


def _mm_body_acc(g_ref, f_ref, we_ref, wo_ref, wf_ref, b_ref, _prev_ref, o_ref):
    _mm_body(g_ref, f_ref, we_ref, wo_ref, wf_ref, b_ref, o_ref)


def _tc_project_chunk(k, g3, ft, we, wo, wf_t, b2, prev):
    grid = (NB_K, SEQ_LEN // SB)
    in_specs = [
        pl.BlockSpec((SB, QUAR, HIDDEN_DIM), lambda ib, isq: (isq, ib, 0)),
        pl.BlockSpec(
            (FEAT_EXTRA, SB, BB), lambda ib, isq, k=k: (0, isq, k * NB_K + ib)
        ),
        pl.BlockSpec((ROW_I32, HIDDEN_DIM), lambda ib, isq: (0, 0)),
        pl.BlockSpec((ROW_I32, HIDDEN_DIM), lambda ib, isq: (0, 0)),
        pl.BlockSpec((FEAT_EXTRA, HIDDEN_DIM), lambda ib, isq: (0, 0)),
        pl.BlockSpec((1, HIDDEN_DIM), lambda ib, isq: (0, 0)),
    ]
    out_spec = pl.BlockSpec(
        (BB, SB, HIDDEN_DIM), lambda ib, isq, k=k: (k * NB_K + ib, isq, 0)
    )
    out_shape = jax.ShapeDtypeStruct((BATCH, SEQ_LEN, HIDDEN_DIM), jnp.float32)
    if prev is None:
        return pl.pallas_call(
            _mm_body,
            grid=grid,
            in_specs=in_specs,
            out_specs=out_spec,
            out_shape=out_shape,
        )(g3, ft, we, wo, wf_t, b2)
    return pl.pallas_call(
        _mm_body_acc,
        grid=grid,
        in_specs=in_specs + [pl.BlockSpec(memory_space=pl.ANY)],
        out_specs=out_spec,
        out_shape=out_shape,
        input_output_aliases={6: 0},
    )(g3, ft, we, wo, wf_t, b2, prev)


@jax.jit
def _run(ids2, ft, table_p, we, wo, wf_t, b2):
    gs = [_sc_gather(k, ids2, table_p) for k in range(K)]
    out = None
    for k in range(K):
        g3 = gs[k].reshape(SEQ_LEN, BPK // 4, HIDDEN_DIM)
        out = _tc_project_chunk(k, g3, ft, we, wo, wf_t, b2, out)
    return out


def kernel(card_ids, card_features, table, W, b):
    ids2 = card_ids.T.astype(jnp.int32)
    tt = table.T
    lo_r = tt[:ROW_I32].astype(jnp.bfloat16).astype(jnp.float32)
    hi_r = tt[ROW_I32:].astype(jnp.bfloat16).astype(jnp.float32)
    words = (
        (lax.bitcast_convert_type(lo_r, jnp.int32) >> 16) & 0xFFFF
    ) | (lax.bitcast_convert_type(hi_r, jnp.int32) & (-65536))
    table_p = words.T
    ft = jnp.transpose(card_features, (2, 1, 0))
    wid_t = W[:, :CARD_ID_DIM].T
    we = wid_t[:ROW_I32]
    wo = wid_t[ROW_I32:]
    wf_t = W[:, CARD_ID_DIM:].T
    b2 = b.reshape(1, HIDDEN_DIM)
    return _run(ids2, ft, table_p, we, wo, wf_t, b2)


# speedup vs baseline: 1.3575x; 1.0276x over previous
import jax
import jax.numpy as jnp
from jax import lax
from jax.experimental import pallas as pl
from jax.experimental.pallas import tpu as pltpu
from jax.experimental.pallas import tpu_sc as plsc

NUM_CARDS = 100000
CARD_ID_DIM = 64
ROW_I32 = CARD_ID_DIM // 2
HIDDEN_DIM = 128
BATCH = 4096
SEQ_LEN = 200
FEAT_EXTRA = 11
TOTAL = BATCH * SEQ_LEN

NC = 2
NS = 16
NW = NC * NS
CHUNK = 128

K = 4
NB = BATCH // 512
NB_K = NB // K
BPK = BATCH // K
ROWS_K = TOTAL // K
PER_W = ROWS_K // NW
NCHUNK = PER_W // CHUNK
SROWS = 8

BB = 512
SB = 40
QUAR = BB // 4


def _make_gather_body(kofs):
    def _gather_body(ids_hbm, table_hbm, out_hbm, sids, idx_v, rows_v, sem_g):
        _gather_impl(kofs, ids_hbm, table_hbm, out_hbm, sids, idx_v, rows_v, sem_g)
    return _gather_body


def _gather_impl(kofs, ids_hbm, table_hbm, out_hbm, sids, idx_v, rows_v, sem_g):
    wid = lax.axis_index("s") * NC + lax.axis_index("c")
    base = wid * PER_W
    s_lo = jnp.minimum(base >> 10, SEQ_LEN - SROWS)

    pltpu.sync_copy(ids_hbm.at[pl.ds(s_lo, SROWS), pl.ds(kofs, BPK)], sids)

    lanes = lax.iota(jnp.int32, 16)
    phase = lanes & 3
    idx_base = lanes >> 2
    dn = lax.GatherDimensionNumbers(
        offset_dims=(), collapsed_slice_dims=(0,), start_index_map=(0,)
    )

    def rgather(vec, idx):
        return lax.gather(
            vec, idx[:, None], dn, (1,),
            mode=lax.GatherScatterMode.PROMISE_IN_BOUNDS,
        )

    def build(i, _):
        flat = base + i * CHUNK
        srel = (flat >> 10) - s_lo
        rem_base = flat & 1023
        ibl = rem_base >> 9
        j0 = (rem_base >> 2) & 127
        for grp in range(2):
            srcs = [
                sids[srel, pl.ds(ibl * 512 + h * 128 + j0 + grp * 16, 16)]
                for h in range(4)
            ]
            for q in range(4):
                idxq = idx_base + q * 4
                g0 = rgather(srcs[0], idxq)
                g1 = rgather(srcs[1], idxq)
                g2 = rgather(srcs[2], idxq)
                g3 = rgather(srcs[3], idxq)
                vals = jnp.where(
                    phase == 0, g0,
                    jnp.where(phase == 1, g1, jnp.where(phase == 2, g2, g3)),
                )
                idx_v[i, pl.ds(grp * 64 + q * 16, 16)] = vals
        return ()

    lax.fori_loop(0, NCHUNK, build, (), unroll=False)

    def body(i, _):
        pltpu.async_copy(table_hbm.at[idx_v.at[i]], rows_v, sem_g).wait()
        pltpu.sync_copy(rows_v, out_hbm.at[pl.ds(base + i * CHUNK, CHUNK)])
        return ()

    lax.fori_loop(0, NCHUNK, body, (), unroll=False)


def _sc_gather(k, ids2, table_p):
    mesh = plsc.VectorSubcoreMesh(
        core_axis_name="c", subcore_axis_name="s", num_cores=NC, num_subcores=NS
    )
    return pl.kernel(
        _make_gather_body(k * BPK),
        out_type=jax.ShapeDtypeStruct((ROWS_K, ROW_I32), jnp.int32),
        mesh=mesh,
        compiler_params=pltpu.CompilerParams(use_tc_tiling_on_sc=False),
        scratch_types=[
            pltpu.VMEM((SROWS, BPK), jnp.int32),
            pltpu.VMEM((NCHUNK, CHUNK), jnp.int32),
            pltpu.VMEM((CHUNK, ROW_I32), jnp.int32),
            pltpu.SemaphoreType.DMA,
        ],
    )(ids2, table_p)


def _mm_body(g_ref, f_ref, we_ref, wo_ref, wf_ref, b_ref, o_ref):
    bvec = b_ref[...]
    we = we_ref[...]
    wo = wo_ref[...]
    wf = wf_ref[...]
    for s in range(SB):
        gi = g_ref[s]
        lo = lax.bitcast_convert_type(gi << 16, jnp.float32)
        hi = lax.bitcast_convert_type(gi & (-65536), jnp.float32)
        fts = f_ref[:, s, :]
        of = lax.dot_general(
            fts, wf,
            dimension_numbers=(((0,), (0,)), ((), ())),
            preferred_element_type=jnp.float32,
        )
        for h in range(4):
            sl = slice(h * ROW_I32, (h + 1) * ROW_I32)
            oh = jnp.dot(lo[:, sl], we, preferred_element_type=jnp.float32)
            oh = oh + jnp.dot(hi[:, sl], wo, preferred_element_type=jnp.float32)
            o_ref[h * QUAR:(h + 1) * QUAR, s, :] = (
                oh + of[h * QUAR:(h + 1) * QUAR] + bvec
            )


def _mm_body_acc(g_ref, f_ref, we_ref, wo_ref, wf_ref, b_ref, _prev_ref, o_ref):
    _mm_body(g_ref, f_ref, we_ref, wo_ref, wf_ref, b_ref, o_ref)


def _tc_project_chunk(k, g3, ft, we, wo, wf_t, b2, prev):
    grid = (NB_K, SEQ_LEN // SB)
    in_specs = [
        pl.BlockSpec((SB, QUAR, HIDDEN_DIM), lambda ib, isq: (isq, ib, 0)),
        pl.BlockSpec(
            (FEAT_EXTRA, SB, BB), lambda ib, isq, k=k: (0, isq, k * NB_K + ib)
        ),
        pl.BlockSpec((ROW_I32, HIDDEN_DIM), lambda ib, isq: (0, 0)),
        pl.BlockSpec((ROW_I32, HIDDEN_DIM), lambda ib, isq: (0, 0)),
        pl.BlockSpec((FEAT_EXTRA, HIDDEN_DIM), lambda ib, isq: (0, 0)),
        pl.BlockSpec((1, HIDDEN_DIM), lambda ib, isq: (0, 0)),
    ]
    out_spec = pl.BlockSpec(
        (BB, SB, HIDDEN_DIM), lambda ib, isq, k=k: (k * NB_K + ib, isq, 0)
    )
    out_shape = jax.ShapeDtypeStruct((BATCH, SEQ_LEN, HIDDEN_DIM), jnp.float32)
    if prev is None:
        return pl.pallas_call(
            _mm_body,
            grid=grid,
            in_specs=in_specs,
            out_specs=out_spec,
            out_shape=out_shape,
        )(g3, ft, we, wo, wf_t, b2)
    return pl.pallas_call(
        _mm_body_acc,
        grid=grid,
        in_specs=in_specs + [pl.BlockSpec(memory_space=pl.ANY)],
        out_specs=out_spec,
        out_shape=out_shape,
        input_output_aliases={6: 0},
    )(g3, ft, we, wo, wf_t, b2, prev)


@jax.jit
def _run(ids2, ft, table_p, we, wo, wf_t, b2):
    gs = [_sc_gather(k, ids2, table_p) for k in range(K)]
    out = None
    for k in range(K):
        g3 = gs[k].reshape(SEQ_LEN, BPK // 4, HIDDEN_DIM)
        out = _tc_project_chunk(k, g3, ft, we, wo, wf_t, b2, out)
    return out


def kernel(card_ids, card_features, table, W, b):
    ids2 = card_ids.T.astype(jnp.int32)
    tt = table.T
    ulo = lax.bitcast_convert_type(tt[:ROW_I32], jnp.int32)
    uhi = lax.bitcast_convert_type(tt[ROW_I32:], jnp.int32)
    rlo = ulo + 0x7FFF + ((ulo >> 16) & 1)
    rhi = uhi + 0x7FFF + ((uhi >> 16) & 1)
    words = ((rlo >> 16) & 0xFFFF) | (rhi & (-65536))
    table_p = words.T
    ft = jnp.transpose(card_features, (2, 1, 0))
    wid_t = W[:, :CARD_ID_DIM].T
    we = wid_t[:ROW_I32]
    wo = wid_t[ROW_I32:]
    wf_t = W[:, CARD_ID_DIM:].T
    b2 = b.reshape(1, HIDDEN_DIM)
    return _run(ids2, ft, table_p, we, wo, wf_t, b2)
